# Initial kernel scaffold; baseline (speedup 1.0000x reference)
import jax
import jax.numpy as jnp
from jax.experimental import pallas as pl


def kernel(embeddings, labels):
    raise NotImplementedError("write your pallas kernel here")


# speedup vs baseline: 6.3889x; 6.3889x over previous
import functools

import jax
import jax.numpy as jnp
from jax import lax
from jax.experimental import pallas as pl
from jax.experimental.pallas import tpu as pltpu

_MARGIN = 0.2
_B = 512
_D = 128
_BIG = 1e30
_AB = 8


def _triplet_kernel(emb_ref, lab_ref, loss_ref, cnt_ref, apm_ref, anm_ref):
    e = emb_ref[...]
    labels = lab_ref[...]

    r = jnp.sum(e * e, axis=1, keepdims=True)
    g = jnp.dot(e, e.T, precision=lax.Precision.HIGHEST,
                preferred_element_type=jnp.float32)
    dist = r + r.T - 2.0 * g

    same = labels == labels.T
    row_ids = lax.broadcasted_iota(jnp.int32, (_B, _B), 0)
    col_ids = lax.broadcasted_iota(jnp.int32, (_B, _B), 1)
    diag = row_ids == col_ids
    pos = same & (~diag)
    neg = ~same

    apm_ref[...] = jnp.where(pos, dist + _MARGIN, -_BIG)
    anm_ref[...] = jnp.where(neg, dist, _BIG)

    npos = jnp.sum(pos.astype(jnp.int32), axis=1, keepdims=True)
    nneg = jnp.sum(neg.astype(jnp.int32), axis=1, keepdims=True)
    cnt_ref[...] = jnp.sum(npos * nneg).reshape(1, 1)

    def body(i, acc):
        a0 = i * _AB
        ap_blk = apm_ref[pl.ds(a0, _AB), :]
        an_blk = anm_ref[pl.ds(a0, _AB), :]
        terms = jnp.maximum(ap_blk[:, :, None] - an_blk[:, None, :], 0.0)
        return acc + jnp.sum(terms, axis=1)

    acc = lax.fori_loop(0, _B // _AB, body,
                        jnp.zeros((_AB, _B), jnp.float32))
    loss_ref[...] = jnp.sum(acc).reshape(1, 1)


@jax.jit
def kernel(embeddings, labels):
    labels2d = labels.reshape(_B, 1)
    loss_sum, count = pl.pallas_call(
        _triplet_kernel,
        out_shape=(
            jax.ShapeDtypeStruct((1, 1), jnp.float32),
            jax.ShapeDtypeStruct((1, 1), jnp.int32),
        ),
        scratch_shapes=(
            pltpu.VMEM((_B, _B), jnp.float32),
            pltpu.VMEM((_B, _B), jnp.float32),
        ),
    )(embeddings, labels2d)
    return loss_sum[0, 0] / count[0, 0].astype(jnp.float32)
